# Initial kernel scaffold; baseline (speedup 1.0000x reference)
import jax
import jax.numpy as jnp
from jax.experimental import pallas as pl


def kernel(node_feature, one_order_bias, two_order_bias, W11, a_src11, a_dst11, b11, W21, a_src21, a_dst21, b21, W12, a_src12, a_dst12, b12, W22, a_src22, a_dst22, b22, combine_weight, combine_weight2):
    raise NotImplementedError("write your pallas kernel here")


# speedup vs baseline: 1.5266x; 1.5266x over previous
import jax
import jax.numpy as jnp
from jax.experimental import pallas as pl

SLOPE = 0.25
BLK = 256


def _elu(x):
    return jnp.where(x > 0, x, jnp.exp(x) - 1.0)


def _prep_kernel(x_ref, w_ref, ssrc_ref, sdst_ref, h_ref, fsrc_ref, fdst_ref):
    h = jnp.dot(x_ref[...], w_ref[...], preferred_element_type=jnp.float32)
    h_ref[...] = h
    fsrc_ref[...] = jnp.dot(h, ssrc_ref[...], preferred_element_type=jnp.float32)
    fdst_ref[...] = jnp.dot(h, sdst_ref[...], preferred_element_type=jnp.float32)


def _attn_kernel(heads, mid, bias_ref, h_ref, f1_ref, f2t_ref, b_ref, out_ref):
    bias = bias_ref[...]
    for hd in range(heads):
        z = f1_ref[:, hd:hd + 1] + f2t_ref[hd:hd + 1, :]
        z = jnp.maximum(z, SLOPE * z) + bias
        m = jnp.max(z, axis=1, keepdims=True)
        p = jnp.exp(z - m)
        s = jnp.sum(p, axis=1, keepdims=True)
        o = jnp.dot(p, h_ref[:, hd * mid:(hd + 1) * mid],
                    preferred_element_type=jnp.float32) / s
        out_ref[:, hd * mid:(hd + 1) * mid] = _elu(o + b_ref[:, hd * mid:(hd + 1) * mid])


def _combine_kernel(a_ref, c_ref, w0_ref, w1_ref, v0_ref, v1_ref, out_ref):
    z = (_elu(a_ref[...] * w0_ref[...]) * v0_ref[...]
         + _elu(c_ref[...] * w1_ref[...]) * v1_ref[...])
    m = jnp.max(z, axis=1, keepdims=True)
    zz = z - m
    out_ref[...] = zz - jnp.log(jnp.sum(jnp.exp(zz), axis=1, keepdims=True))


def _scatter_mat(a):
    heads, mid = a.shape
    eye = jnp.eye(heads, dtype=a.dtype)
    return (a[:, :, None] * eye[:, None, :]).reshape(heads * mid, heads)


def _run_prep(x, w, ssrc, sdst):
    n, _ = x.shape
    hc = w.shape[1]
    nh = ssrc.shape[1]
    return pl.pallas_call(
        _prep_kernel,
        out_shape=(
            jax.ShapeDtypeStruct((n, hc), jnp.float32),
            jax.ShapeDtypeStruct((n, nh), jnp.float32),
            jax.ShapeDtypeStruct((n, nh), jnp.float32),
        ),
    )(x, w, ssrc, sdst)


def _run_attn(heads, mid, bias, h, f1, f2t, b):
    n = bias.shape[0]
    grid = (n // BLK,)
    return pl.pallas_call(
        lambda *refs: _attn_kernel(heads, mid, *refs),
        grid=grid,
        in_specs=[
            pl.BlockSpec((BLK, n), lambda i: (i, 0)),
            pl.BlockSpec((n, heads * mid), lambda i: (0, 0)),
            pl.BlockSpec((BLK, heads), lambda i: (i, 0)),
            pl.BlockSpec((heads, n), lambda i: (0, 0)),
            pl.BlockSpec((1, heads * mid), lambda i: (0, 0)),
        ],
        out_specs=pl.BlockSpec((BLK, heads * mid), lambda i: (i, 0)),
        out_shape=jax.ShapeDtypeStruct((n, heads * mid), jnp.float32),
    )(bias, h, f1, f2t, b)


def kernel(node_feature, one_order_bias, two_order_bias,
           W11, a_src11, a_dst11, b11,
           W21, a_src21, a_dst21, b21,
           W12, a_src12, a_dst12, b12,
           W22, a_src22, a_dst22, b22,
           combine_weight, combine_weight2):
    n = node_feature.shape[0]
    heads, mid = a_src11.shape
    hc = heads * mid
    out_ch = W12.shape[1]

    wcat = jnp.concatenate([W11, W21], axis=1)
    ssrc = jnp.zeros((2 * hc, 2 * heads), jnp.float32)
    ssrc = ssrc.at[:hc, :heads].set(_scatter_mat(a_src11))
    ssrc = ssrc.at[hc:, heads:].set(_scatter_mat(a_src21))
    sdst = jnp.zeros((2 * hc, 2 * heads), jnp.float32)
    sdst = sdst.at[:hc, :heads].set(_scatter_mat(a_dst11))
    sdst = sdst.at[hc:, heads:].set(_scatter_mat(a_dst21))
    h1, fsrc1, fdst1 = _run_prep(node_feature, wcat, ssrc, sdst)
    fdst1t = fdst1.T

    ns11 = _run_attn(heads, mid, one_order_bias, h1[:, :hc],
                     fsrc1[:, :heads], fdst1t[:heads, :], b11.reshape(1, hc))
    ns21 = _run_attn(heads, mid, two_order_bias, h1[:, hc:],
                     fsrc1[:, heads:], fdst1t[heads:, :], b21.reshape(1, hc))

    ns_cat = jnp.concatenate([ns11, ns21], axis=1)
    w2 = jnp.zeros((2 * hc, 2 * out_ch), jnp.float32)
    w2 = w2.at[:hc, :out_ch].set(W12)
    w2 = w2.at[hc:, out_ch:].set(W22)
    s2src = jnp.zeros((2 * out_ch, 2), jnp.float32)
    s2src = s2src.at[:out_ch, 0].set(a_src12[0])
    s2src = s2src.at[out_ch:, 1].set(a_src22[0])
    s2dst = jnp.zeros((2 * out_ch, 2), jnp.float32)
    s2dst = s2dst.at[:out_ch, 0].set(a_dst12[0])
    s2dst = s2dst.at[out_ch:, 1].set(a_dst22[0])
    h2, fsrc2, fdst2 = _run_prep(ns_cat, w2, s2src, s2dst)
    fdst2t = fdst2.T

    ns12 = _run_attn(1, out_ch, one_order_bias, h2[:, :out_ch],
                     fsrc2[:, 0:1], fdst2t[0:1, :], b12.reshape(1, out_ch))
    ns22 = _run_attn(1, out_ch, two_order_bias, h2[:, out_ch:],
                     fsrc2[:, 1:2], fdst2t[1:2, :], b22.reshape(1, out_ch))

    w0 = combine_weight[0, :, 0].reshape(1, out_ch)
    w1 = combine_weight[0, :, 1].reshape(1, out_ch)
    v0 = combine_weight2[0, :, 0].reshape(1, out_ch)
    v1 = combine_weight2[0, :, 1].reshape(1, out_ch)
    return pl.pallas_call(
        _combine_kernel,
        out_shape=jax.ShapeDtypeStruct((n, out_ch), jnp.float32),
    )(ns12, ns22, w0, w1, v0, v1)


# speedup vs baseline: 3.2964x; 2.1593x over previous
import jax
import jax.numpy as jnp
from jax.experimental import pallas as pl
from jax.experimental.pallas import tpu as pltpu

SLOPE = 0.25
BLK = 256
LOG2E = 1.4426950408889634


def _elu(x):
    return jnp.where(x > 0, x, jnp.exp(x) - 1.0)


def _eye(k):
    r = jax.lax.broadcasted_iota(jnp.int32, (k, k), 0)
    c = jax.lax.broadcasted_iota(jnp.int32, (k, k), 1)
    return (r == c).astype(jnp.float32)


def _prep(hc, nh, x_ref, w_ref, ssrc_ref, sdst_ref,
          a1_ref, a4_ref, b1t_ref, b4t_ref, haug1_ref, haug2_ref):
    h = jnp.dot(x_ref[...].astype(jnp.bfloat16), w_ref[...],
                preferred_element_type=jnp.float32)
    fsrc = jnp.dot(h, ssrc_ref[...], preferred_element_type=jnp.float32)
    fdst = jnp.dot(h, sdst_ref[...], preferred_element_type=jnp.float32)
    fdstt = jax.lax.dot_general(_eye(nh), fdst, (((1,), (1,)), ((), ())),
                                preferred_element_type=jnp.float32)
    a1_ref[...] = jnp.exp2(fsrc).astype(jnp.bfloat16)
    a4_ref[...] = jnp.exp2(0.25 * fsrc).astype(jnp.bfloat16)
    b1t_ref[...] = jnp.exp2(fdstt).astype(jnp.bfloat16)
    b4t_ref[...] = jnp.exp2(0.25 * fdstt).astype(jnp.bfloat16)
    hb = h.astype(jnp.bfloat16)
    haug1_ref[:, :hc] = hb[:, :hc]
    haug1_ref[:, hc:] = jnp.ones_like(haug1_ref[:, hc:])
    haug2_ref[:, :hc] = hb[:, hc:]
    haug2_ref[:, hc:] = jnp.ones_like(haug2_ref[:, hc:])


def _branch8(heads, mid, row0, mask, haug_ref, a1_ref, a4_ref,
             b1t_ref, b4t_ref, b_ref, out_ref, a_ref, col, ocol):
    hc = heads * mid
    for hd in range(heads):
        c = col + hd
        p1 = a1_ref[pl.ds(row0, BLK), c:c + 1] * b1t_ref[c:c + 1, :]
        p4 = a4_ref[pl.ds(row0, BLK), c:c + 1] * b4t_ref[c:c + 1, :]
        a_ref[hd * BLK:(hd + 1) * BLK, :] = jnp.maximum(p1, p4) * mask
    ob = jnp.dot(a_ref[...], haug_ref[...], preferred_element_type=jnp.float32)
    for hd in range(heads):
        o = ob[hd * BLK:(hd + 1) * BLK, hd * mid:(hd + 1) * mid]
        s = ob[hd * BLK:(hd + 1) * BLK, hc:hc + 1]
        out_ref[:, ocol + hd * mid:ocol + (hd + 1) * mid] = _elu(
            o / s + b_ref[:, ocol + hd * mid:ocol + (hd + 1) * mid])


def _attn_l1_kernel(heads, mid, x_ref, w_ref, ssrc_ref, sdst_ref,
                    bias1_ref, bias2_ref, b_ref,
                    out_ref, mask1_ref, mask2_ref,
                    a_ref, a1_ref, a4_ref, b1t_ref, b4t_ref,
                    haug1_ref, haug2_ref):
    hc = heads * mid
    i = pl.program_id(0)

    @pl.when(i == 0)
    def _():
        _prep(hc, 2 * heads, x_ref, w_ref, ssrc_ref, sdst_ref,
              a1_ref, a4_ref, b1t_ref, b4t_ref, haug1_ref, haug2_ref)

    row0 = i * BLK
    m1 = (bias1_ref[...] > -1e8).astype(jnp.bfloat16)
    mask1_ref[...] = m1
    _branch8(heads, mid, row0, m1, haug1_ref, a1_ref, a4_ref, b1t_ref,
             b4t_ref, b_ref, out_ref, a_ref, 0, 0)
    m2 = (bias2_ref[...] > -1e8).astype(jnp.bfloat16)
    mask2_ref[...] = m2
    _branch8(heads, mid, row0, m2, haug2_ref, a1_ref, a4_ref, b1t_ref,
             b4t_ref, b_ref, out_ref, a_ref, heads, hc)


def _branch1(ch, row0, mask, haug_ref, a1_ref, a4_ref, b1t_ref, b4t_ref,
             b_ref, col):
    p1 = a1_ref[pl.ds(row0, BLK), col:col + 1] * b1t_ref[col:col + 1, :]
    p4 = a4_ref[pl.ds(row0, BLK), col:col + 1] * b4t_ref[col:col + 1, :]
    p = jnp.maximum(p1, p4) * mask
    ob = jnp.dot(p, haug_ref[...], preferred_element_type=jnp.float32)
    return _elu(ob[:, :ch] / ob[:, ch:ch + 1] + b_ref[col:col + 1, :])


def _attn_l2_kernel(ch, ns_ref, w_ref, ssrc_ref, sdst_ref,
                    mask1_ref, mask2_ref, b_ref, cw_ref, out_ref,
                    a1_ref, a4_ref, b1t_ref, b4t_ref,
                    haug1_ref, haug2_ref):
    i = pl.program_id(0)

    @pl.when(i == 0)
    def _():
        _prep(ch, 2, ns_ref, w_ref, ssrc_ref, sdst_ref,
              a1_ref, a4_ref, b1t_ref, b4t_ref, haug1_ref, haug2_ref)

    row0 = i * BLK
    ns12 = _branch1(ch, row0, mask1_ref[...], haug1_ref, a1_ref, a4_ref,
                    b1t_ref, b4t_ref, b_ref, 0)
    ns22 = _branch1(ch, row0, mask2_ref[...], haug2_ref, a1_ref, a4_ref,
                    b1t_ref, b4t_ref, b_ref, 1)
    z = (_elu(ns12 * cw_ref[0:1, :]) * cw_ref[2:3, :]
         + _elu(ns22 * cw_ref[1:2, :]) * cw_ref[3:4, :])
    m = jnp.max(z, axis=1, keepdims=True)
    zz = z - m
    out_ref[...] = zz - jnp.log(jnp.sum(jnp.exp(zz), axis=1, keepdims=True))


def _scatter_mat(a):
    heads, mid = a.shape
    eye = jnp.eye(heads, dtype=a.dtype)
    return (a[:, :, None] * eye[:, None, :]).reshape(heads * mid, heads)


def kernel(node_feature, one_order_bias, two_order_bias,
           W11, a_src11, a_dst11, b11,
           W21, a_src21, a_dst21, b21,
           W12, a_src12, a_dst12, b12,
           W22, a_src22, a_dst22, b22,
           combine_weight, combine_weight2):
    n = node_feature.shape[0]
    in_ch = node_feature.shape[1]
    heads, mid = a_src11.shape
    hc = heads * mid
    out_ch = W12.shape[1]
    f32 = jnp.float32
    bf16 = jnp.bfloat16

    wcat = jnp.concatenate([W11, W21], axis=1).astype(bf16)
    ssrc = jnp.zeros((2 * hc, 2 * heads), f32)
    ssrc = ssrc.at[:hc, :heads].set(_scatter_mat(a_src11))
    ssrc = ssrc.at[hc:, heads:].set(_scatter_mat(a_src21))
    sdst = jnp.zeros((2 * hc, 2 * heads), f32)
    sdst = sdst.at[:hc, :heads].set(_scatter_mat(a_dst11))
    sdst = sdst.at[hc:, heads:].set(_scatter_mat(a_dst21))
    bcat1 = jnp.concatenate([b11, b21]).reshape(1, 2 * hc)

    full = lambda shape: pl.BlockSpec(shape, lambda i: tuple(0 for _ in shape))
    rows = lambda shape: pl.BlockSpec(shape, lambda i: (i, 0))

    ns_cat, mask1, mask2 = pl.pallas_call(
        lambda *refs: _attn_l1_kernel(heads, mid, *refs),
        grid=(n // BLK,),
        in_specs=[
            full((n, in_ch)),
            full((in_ch, 2 * hc)),
            full((2 * hc, 2 * heads)),
            full((2 * hc, 2 * heads)),
            rows((BLK, n)),
            rows((BLK, n)),
            full((1, 2 * hc)),
        ],
        out_specs=[rows((BLK, 2 * hc)), rows((BLK, n)), rows((BLK, n))],
        out_shape=[
            jax.ShapeDtypeStruct((n, 2 * hc), f32),
            jax.ShapeDtypeStruct((n, n), bf16),
            jax.ShapeDtypeStruct((n, n), bf16),
        ],
        scratch_shapes=[
            pltpu.VMEM((heads * BLK, n), bf16),
            pltpu.VMEM((n, 2 * heads), bf16),
            pltpu.VMEM((n, 2 * heads), bf16),
            pltpu.VMEM((2 * heads, n), bf16),
            pltpu.VMEM((2 * heads, n), bf16),
            pltpu.VMEM((n, hc + 1), bf16),
            pltpu.VMEM((n, hc + 1), bf16),
        ],
    )(node_feature, wcat, ssrc * LOG2E, sdst * LOG2E,
      one_order_bias, two_order_bias, bcat1)

    w2 = jnp.zeros((2 * hc, 2 * out_ch), f32)
    w2 = w2.at[:hc, :out_ch].set(W12)
    w2 = w2.at[hc:, out_ch:].set(W22)
    s2src = jnp.zeros((2 * out_ch, 2), f32)
    s2src = s2src.at[:out_ch, 0].set(a_src12[0])
    s2src = s2src.at[out_ch:, 1].set(a_src22[0])
    s2dst = jnp.zeros((2 * out_ch, 2), f32)
    s2dst = s2dst.at[:out_ch, 0].set(a_dst12[0])
    s2dst = s2dst.at[out_ch:, 1].set(a_dst22[0])
    bcat2 = jnp.stack([b12, b22])
    cw = jnp.concatenate([combine_weight[0].T, combine_weight2[0].T])

    return pl.pallas_call(
        lambda *refs: _attn_l2_kernel(out_ch, *refs),
        grid=(n // BLK,),
        in_specs=[
            full((n, 2 * hc)),
            full((2 * hc, 2 * out_ch)),
            full((2 * out_ch, 2)),
            full((2 * out_ch, 2)),
            rows((BLK, n)),
            rows((BLK, n)),
            full((2, out_ch)),
            full((4, out_ch)),
        ],
        out_specs=rows((BLK, out_ch)),
        out_shape=jax.ShapeDtypeStruct((n, out_ch), f32),
        scratch_shapes=[
            pltpu.VMEM((n, 2), bf16),
            pltpu.VMEM((n, 2), bf16),
            pltpu.VMEM((2, n), bf16),
            pltpu.VMEM((2, n), bf16),
            pltpu.VMEM((n, out_ch + 1), bf16),
            pltpu.VMEM((n, out_ch + 1), bf16),
        ],
    )(ns_cat, w2.astype(bf16), s2src * LOG2E, s2dst * LOG2E,
      mask1, mask2, bcat2, cw)


# speedup vs baseline: 3.4221x; 1.0381x over previous
import jax
import jax.numpy as jnp
from jax.experimental import pallas as pl
from jax.experimental.pallas import tpu as pltpu

SLOPE = 0.25
BLK = 256
LOG2E = 1.4426950408889634


def _elu(x):
    return jnp.where(x > 0, x, jnp.exp(x) - 1.0)


def _eye(k):
    r = jax.lax.broadcasted_iota(jnp.int32, (k, k), 0)
    c = jax.lax.broadcasted_iota(jnp.int32, (k, k), 1)
    return (r == c).astype(jnp.float32)


def _prep(hc, nh, x_ref, w_ref, ssrc_ref, sdst_ref,
          a1_ref, a4_ref, b1t_ref, b4t_ref, haug1_ref, haug2_ref):
    h = jnp.dot(x_ref[...].astype(jnp.bfloat16), w_ref[...],
                preferred_element_type=jnp.float32)
    fsrc = jnp.dot(h, ssrc_ref[...], preferred_element_type=jnp.float32)
    fdst = jnp.dot(h, sdst_ref[...], preferred_element_type=jnp.float32)
    fdstt = jax.lax.dot_general(_eye(nh), fdst, (((1,), (1,)), ((), ())),
                                preferred_element_type=jnp.float32)
    a1_ref[...] = jnp.exp2(fsrc).astype(jnp.bfloat16)
    a4_ref[...] = jnp.exp2(0.25 * fsrc).astype(jnp.bfloat16)
    b1t_ref[...] = jnp.exp2(fdstt).astype(jnp.bfloat16)
    b4t_ref[...] = jnp.exp2(0.25 * fdstt).astype(jnp.bfloat16)
    hb = h.astype(jnp.bfloat16)
    haug1_ref[:, :hc] = hb[:, :hc]
    haug1_ref[:, hc:] = jnp.ones_like(haug1_ref[:, hc:])
    haug2_ref[:, :hc] = hb[:, hc:]
    haug2_ref[:, hc:] = jnp.ones_like(haug2_ref[:, hc:])


def _branch8(heads, mid, row0, mask, haug_ref, a1_ref, a4_ref,
             b1t_ref, b4t_ref, b_ref, ns_ref, a_ref, col, ocol):
    hc = heads * mid
    for hd in range(heads):
        c = col + hd
        p1 = a1_ref[pl.ds(row0, BLK), c:c + 1] * b1t_ref[c:c + 1, :]
        p4 = a4_ref[pl.ds(row0, BLK), c:c + 1] * b4t_ref[c:c + 1, :]
        a_ref[hd * BLK:(hd + 1) * BLK, :] = jnp.maximum(p1, p4) * mask
    ob = jnp.dot(a_ref[...], haug_ref[...], preferred_element_type=jnp.float32)
    for hd in range(heads):
        o = ob[hd * BLK:(hd + 1) * BLK, hd * mid:(hd + 1) * mid]
        s = ob[hd * BLK:(hd + 1) * BLK, hc:hc + 1]
        ns_ref[pl.ds(row0, BLK), ocol + hd * mid:ocol + (hd + 1) * mid] = _elu(
            o / s + b_ref[:, ocol + hd * mid:ocol + (hd + 1) * mid])


def _branch1(ch, row0, mask, haug_ref, a1_ref, a4_ref, b1t_ref, b4t_ref,
             b_ref, col):
    p1 = a1_ref[pl.ds(row0, BLK), col:col + 1] * b1t_ref[col:col + 1, :]
    p4 = a4_ref[pl.ds(row0, BLK), col:col + 1] * b4t_ref[col:col + 1, :]
    p = jnp.maximum(p1, p4) * mask
    ob = jnp.dot(p, haug_ref[...], preferred_element_type=jnp.float32)
    return _elu(ob[:, :ch] / ob[:, ch:ch + 1] + b_ref[col:col + 1, :])


def _mega_kernel(heads, mid, out_ch,
                 x_ref, wcat_ref, ssrc_ref, sdst_ref,
                 w2_ref, s2src_ref, s2dst_ref,
                 bias1_ref, bias2_ref, bcat1_ref, bcat2_ref, cw_ref,
                 out_ref,
                 a_ref, mask1_s, mask2_s, ns_s,
                 a1_s, a4_s, b1t_s, b4t_s, haug1_s, haug2_s,
                 c1_s, c4_s, d1t_s, d4t_s, gaug1_s, gaug2_s):
    hc = heads * mid
    p = pl.program_id(0)
    i = pl.program_id(1)
    row0 = i * BLK

    @pl.when(jnp.logical_and(p == 0, i == 0))
    def _():
        _prep(hc, 2 * heads, x_ref, wcat_ref, ssrc_ref, sdst_ref,
              a1_s, a4_s, b1t_s, b4t_s, haug1_s, haug2_s)

    @pl.when(p == 0)
    def _():
        m1 = (bias1_ref[...] > -1e8).astype(jnp.bfloat16)
        mask1_s[pl.ds(row0, BLK), :] = m1
        _branch8(heads, mid, row0, m1, haug1_s, a1_s, a4_s, b1t_s, b4t_s,
                 bcat1_ref, ns_s, a_ref, 0, 0)
        m2 = (bias2_ref[...] > -1e8).astype(jnp.bfloat16)
        mask2_s[pl.ds(row0, BLK), :] = m2
        _branch8(heads, mid, row0, m2, haug2_s, a1_s, a4_s, b1t_s, b4t_s,
                 bcat1_ref, ns_s, a_ref, heads, hc)

    @pl.when(jnp.logical_and(p == 1, i == 0))
    def _():
        _prep(out_ch, 2, ns_s, w2_ref, s2src_ref, s2dst_ref,
              c1_s, c4_s, d1t_s, d4t_s, gaug1_s, gaug2_s)

    @pl.when(p == 1)
    def _():
        m1 = mask1_s[pl.ds(row0, BLK), :]
        m2 = mask2_s[pl.ds(row0, BLK), :]
        ns12 = _branch1(out_ch, row0, m1, gaug1_s, c1_s, c4_s,
                        d1t_s, d4t_s, bcat2_ref, 0)
        ns22 = _branch1(out_ch, row0, m2, gaug2_s, c1_s, c4_s,
                        d1t_s, d4t_s, bcat2_ref, 1)
        z = (_elu(ns12 * cw_ref[0:1, :]) * cw_ref[2:3, :]
             + _elu(ns22 * cw_ref[1:2, :]) * cw_ref[3:4, :])
        m = jnp.max(z, axis=1, keepdims=True)
        zz = z - m
        out_ref[...] = zz - jnp.log(jnp.sum(jnp.exp(zz), axis=1, keepdims=True))


def _scatter_mat(a):
    heads, mid = a.shape
    eye = jnp.eye(heads, dtype=a.dtype)
    return (a[:, :, None] * eye[:, None, :]).reshape(heads * mid, heads)


def kernel(node_feature, one_order_bias, two_order_bias,
           W11, a_src11, a_dst11, b11,
           W21, a_src21, a_dst21, b21,
           W12, a_src12, a_dst12, b12,
           W22, a_src22, a_dst22, b22,
           combine_weight, combine_weight2):
    n = node_feature.shape[0]
    in_ch = node_feature.shape[1]
    heads, mid = a_src11.shape
    hc = heads * mid
    out_ch = W12.shape[1]
    f32 = jnp.float32
    bf16 = jnp.bfloat16

    wcat = jnp.concatenate([W11, W21], axis=1).astype(bf16)
    ssrc = jnp.zeros((2 * hc, 2 * heads), f32)
    ssrc = ssrc.at[:hc, :heads].set(_scatter_mat(a_src11))
    ssrc = ssrc.at[hc:, heads:].set(_scatter_mat(a_src21))
    sdst = jnp.zeros((2 * hc, 2 * heads), f32)
    sdst = sdst.at[:hc, :heads].set(_scatter_mat(a_dst11))
    sdst = sdst.at[hc:, heads:].set(_scatter_mat(a_dst21))
    bcat1 = jnp.concatenate([b11, b21]).reshape(1, 2 * hc)

    w2 = jnp.zeros((2 * hc, 2 * out_ch), f32)
    w2 = w2.at[:hc, :out_ch].set(W12)
    w2 = w2.at[hc:, out_ch:].set(W22)
    s2src = jnp.zeros((2 * out_ch, 2), f32)
    s2src = s2src.at[:out_ch, 0].set(a_src12[0])
    s2src = s2src.at[out_ch:, 1].set(a_src22[0])
    s2dst = jnp.zeros((2 * out_ch, 2), f32)
    s2dst = s2dst.at[:out_ch, 0].set(a_dst12[0])
    s2dst = s2dst.at[out_ch:, 1].set(a_dst22[0])
    bcat2 = jnp.stack([b12, b22])
    cw = jnp.concatenate([combine_weight[0].T, combine_weight2[0].T])

    full = lambda shape: pl.BlockSpec(shape, lambda p, i: tuple(0 for _ in shape))
    rows = lambda shape: pl.BlockSpec(shape, lambda p, i: (i, 0))
    bias_rows = pl.BlockSpec((BLK, n), lambda p, i: ((1 - p) * i, 0))

    return pl.pallas_call(
        lambda *refs: _mega_kernel(heads, mid, out_ch, *refs),
        grid=(2, n // BLK),
        in_specs=[
            full((n, in_ch)),
            full((in_ch, 2 * hc)),
            full((2 * hc, 2 * heads)),
            full((2 * hc, 2 * heads)),
            full((2 * hc, 2 * out_ch)),
            full((2 * out_ch, 2)),
            full((2 * out_ch, 2)),
            bias_rows,
            bias_rows,
            full((1, 2 * hc)),
            full((2, out_ch)),
            full((4, out_ch)),
        ],
        out_specs=rows((BLK, out_ch)),
        out_shape=jax.ShapeDtypeStruct((n, out_ch), f32),
        scratch_shapes=[
            pltpu.VMEM((heads * BLK, n), bf16),
            pltpu.VMEM((n, n), bf16),
            pltpu.VMEM((n, n), bf16),
            pltpu.VMEM((n, 2 * hc), f32),
            pltpu.VMEM((n, 2 * heads), bf16),
            pltpu.VMEM((n, 2 * heads), bf16),
            pltpu.VMEM((2 * heads, n), bf16),
            pltpu.VMEM((2 * heads, n), bf16),
            pltpu.VMEM((n, hc + 1), bf16),
            pltpu.VMEM((n, hc + 1), bf16),
            pltpu.VMEM((n, 2), bf16),
            pltpu.VMEM((n, 2), bf16),
            pltpu.VMEM((2, n), bf16),
            pltpu.VMEM((2, n), bf16),
            pltpu.VMEM((n, out_ch + 1), bf16),
            pltpu.VMEM((n, out_ch + 1), bf16),
        ],
    )(node_feature, wcat, ssrc * LOG2E, sdst * LOG2E,
      w2.astype(bf16), s2src * LOG2E, s2dst * LOG2E,
      one_order_bias, two_order_bias, bcat1, bcat2, cw)


# speedup vs baseline: 3.8028x; 1.1112x over previous
import jax
import jax.numpy as jnp
from jax.experimental import pallas as pl
from jax.experimental.pallas import tpu as pltpu

SLOPE = 0.25
BLK = 256
LOG2E = 1.4426950408889634


def _elu(x):
    return jnp.where(x > 0, x, jnp.exp(x) - 1.0)


def _eye(k):
    r = jax.lax.broadcasted_iota(jnp.int32, (k, k), 0)
    c = jax.lax.broadcasted_iota(jnp.int32, (k, k), 1)
    return (r == c).astype(jnp.float32)


def _head_summer(hc, heads):
    r = jax.lax.broadcasted_iota(jnp.int32, (hc, heads), 0) // (hc // heads)
    c = jax.lax.broadcasted_iota(jnp.int32, (hc, heads), 1)
    return (r == c).astype(jnp.float32)


def _tr(fd):
    k = fd.shape[1]
    return jax.lax.dot_general(_eye(k), fd, (((1,), (1,)), ((), ())),
                               preferred_element_type=jnp.float32)


def _prep1(heads, hc, x_ref, w1_ref, w2_ref, pr_ref,
           a1_s, a4_s, b1t_s, b4t_s, haug1_s, haug2_s, fd_s):
    bf16 = jnp.bfloat16
    xb = x_ref[...].astype(bf16)
    h1 = jnp.dot(xb, w1_ref[...].astype(bf16), preferred_element_type=jnp.float32)
    h2 = jnp.dot(xb, w2_ref[...].astype(bf16), preferred_element_type=jnp.float32)
    su = _head_summer(hc, heads)
    fs1 = jnp.dot(h1 * pr_ref[0:1, :], su, preferred_element_type=jnp.float32)
    fs2 = jnp.dot(h2 * pr_ref[2:3, :], su, preferred_element_type=jnp.float32)
    fd_s[:, :heads] = jnp.dot(h1 * pr_ref[1:2, :], su,
                              preferred_element_type=jnp.float32)
    fd_s[:, heads:] = jnp.dot(h2 * pr_ref[3:4, :], su,
                              preferred_element_type=jnp.float32)
    a1_s[:, :heads] = jnp.exp2(fs1).astype(bf16)
    a1_s[:, heads:] = jnp.exp2(fs2).astype(bf16)
    a4_s[:, :heads] = jnp.exp2(0.25 * fs1).astype(bf16)
    a4_s[:, heads:] = jnp.exp2(0.25 * fs2).astype(bf16)
    fdt = _tr(fd_s[...])
    b1t_s[...] = jnp.exp2(fdt).astype(bf16)
    b4t_s[...] = jnp.exp2(0.25 * fdt).astype(bf16)
    haug1_s[:, :hc] = h1.astype(bf16)
    haug1_s[:, hc:] = jnp.ones_like(haug1_s[:, hc:])
    haug2_s[:, :hc] = h2.astype(bf16)
    haug2_s[:, hc:] = jnp.ones_like(haug2_s[:, hc:])


def _prep2(hc, ch, ns_s, w1_ref, w2_ref, pr_ref,
           c1_s, c4_s, d1t_s, d4t_s, gaug1_s, gaug2_s, fd2_s):
    bf16 = jnp.bfloat16
    h1 = jnp.dot(ns_s[:, :hc].astype(bf16), w1_ref[...].astype(bf16),
                 preferred_element_type=jnp.float32)
    h2 = jnp.dot(ns_s[:, hc:].astype(bf16), w2_ref[...].astype(bf16),
                 preferred_element_type=jnp.float32)
    ones = jnp.ones((ch, 1), jnp.float32)
    fs1 = jnp.dot(h1 * pr_ref[4:5, :], ones, preferred_element_type=jnp.float32)
    fs2 = jnp.dot(h2 * pr_ref[6:7, :], ones, preferred_element_type=jnp.float32)
    fd2_s[:, 0:1] = jnp.dot(h1 * pr_ref[5:6, :], ones,
                            preferred_element_type=jnp.float32)
    fd2_s[:, 1:2] = jnp.dot(h2 * pr_ref[7:8, :], ones,
                            preferred_element_type=jnp.float32)
    c1_s[:, 0:1] = jnp.exp2(fs1).astype(bf16)
    c1_s[:, 1:2] = jnp.exp2(fs2).astype(bf16)
    c4_s[:, 0:1] = jnp.exp2(0.25 * fs1).astype(bf16)
    c4_s[:, 1:2] = jnp.exp2(0.25 * fs2).astype(bf16)
    fdt = _tr(fd2_s[...])
    d1t_s[...] = jnp.exp2(fdt).astype(bf16)
    d4t_s[...] = jnp.exp2(0.25 * fdt).astype(bf16)
    gaug1_s[:, :ch] = h1.astype(bf16)
    gaug1_s[:, ch:] = jnp.ones_like(gaug1_s[:, ch:])
    gaug2_s[:, :ch] = h2.astype(bf16)
    gaug2_s[:, ch:] = jnp.ones_like(gaug2_s[:, ch:])


def _branch8(heads, mid, row0, mask, haug_ref, a1_ref, a4_ref,
             b1t_ref, b4t_ref, brow, ns_ref, a_ref, col, ocol):
    hc = heads * mid
    for hd in range(heads):
        c = col + hd
        p1 = a1_ref[pl.ds(row0, BLK), c:c + 1] * b1t_ref[c:c + 1, :]
        p4 = a4_ref[pl.ds(row0, BLK), c:c + 1] * b4t_ref[c:c + 1, :]
        a_ref[hd * BLK:(hd + 1) * BLK, :] = jnp.maximum(p1, p4) * mask
    ob = jnp.dot(a_ref[...], haug_ref[...], preferred_element_type=jnp.float32)
    for hd in range(heads):
        o = ob[hd * BLK:(hd + 1) * BLK, hd * mid:(hd + 1) * mid]
        s = ob[hd * BLK:(hd + 1) * BLK, hc:hc + 1]
        ns_ref[pl.ds(row0, BLK), ocol + hd * mid:ocol + (hd + 1) * mid] = _elu(
            o / s + brow[:, hd * mid:(hd + 1) * mid])


def _branch1(ch, row0, mask, haug_ref, c1_ref, c4_ref, d1t_ref, d4t_ref,
             brow, col):
    p1 = c1_ref[pl.ds(row0, BLK), col:col + 1] * d1t_ref[col:col + 1, :]
    p4 = c4_ref[pl.ds(row0, BLK), col:col + 1] * d4t_ref[col:col + 1, :]
    p = jnp.maximum(p1, p4) * mask
    ob = jnp.dot(p, haug_ref[...], preferred_element_type=jnp.float32)
    return _elu(ob[:, :ch] / ob[:, ch:ch + 1] + brow)


def _mega_kernel(heads, mid, out_ch,
                 x_ref, w11_ref, w21_ref, w12_ref, w22_ref,
                 bias1_ref, bias2_ref, pr_ref,
                 out_ref,
                 a_ref, mask1_s, mask2_s, ns_s,
                 a1_s, a4_s, b1t_s, b4t_s, haug1_s, haug2_s,
                 c1_s, c4_s, d1t_s, d4t_s, gaug1_s, gaug2_s,
                 fd_s, fd2_s):
    hc = heads * mid
    p = pl.program_id(0)
    i = pl.program_id(1)
    row0 = i * BLK

    @pl.when(jnp.logical_and(p == 0, i == 0))
    def _():
        _prep1(heads, hc, x_ref, w11_ref, w21_ref, pr_ref,
               a1_s, a4_s, b1t_s, b4t_s, haug1_s, haug2_s, fd_s)

    @pl.when(p == 0)
    def _():
        m1 = (bias1_ref[...] > -1e8).astype(jnp.bfloat16)
        mask1_s[pl.ds(row0, BLK), :] = m1
        _branch8(heads, mid, row0, m1, haug1_s, a1_s, a4_s, b1t_s, b4t_s,
                 pr_ref[8:9, :], ns_s, a_ref, 0, 0)
        m2 = (bias2_ref[...] > -1e8).astype(jnp.bfloat16)
        mask2_s[pl.ds(row0, BLK), :] = m2
        _branch8(heads, mid, row0, m2, haug2_s, a1_s, a4_s, b1t_s, b4t_s,
                 pr_ref[9:10, :], ns_s, a_ref, heads, hc)

    @pl.when(jnp.logical_and(p == 1, i == 0))
    def _():
        _prep2(hc, out_ch, ns_s, w12_ref, w22_ref, pr_ref,
               c1_s, c4_s, d1t_s, d4t_s, gaug1_s, gaug2_s, fd2_s)

    @pl.when(p == 1)
    def _():
        m1 = mask1_s[pl.ds(row0, BLK), :]
        m2 = mask2_s[pl.ds(row0, BLK), :]
        ns12 = _branch1(out_ch, row0, m1, gaug1_s, c1_s, c4_s,
                        d1t_s, d4t_s, pr_ref[10:11, :], 0)
        ns22 = _branch1(out_ch, row0, m2, gaug2_s, c1_s, c4_s,
                        d1t_s, d4t_s, pr_ref[11:12, :], 1)
        z = (_elu(ns12 * pr_ref[12:13, :]) * pr_ref[14:15, :]
             + _elu(ns22 * pr_ref[13:14, :]) * pr_ref[15:16, :])
        m = jnp.max(z, axis=1, keepdims=True)
        zz = z - m
        out_ref[...] = zz - jnp.log(jnp.sum(jnp.exp(zz), axis=1, keepdims=True))


def kernel(node_feature, one_order_bias, two_order_bias,
           W11, a_src11, a_dst11, b11,
           W21, a_src21, a_dst21, b21,
           W12, a_src12, a_dst12, b12,
           W22, a_src22, a_dst22, b22,
           combine_weight, combine_weight2):
    n = node_feature.shape[0]
    in_ch = node_feature.shape[1]
    heads, mid = a_src11.shape
    hc = heads * mid
    out_ch = W12.shape[1]
    f32 = jnp.float32

    params = jnp.concatenate([
        a_src11.reshape(1, hc) * LOG2E,
        a_dst11.reshape(1, hc) * LOG2E,
        a_src21.reshape(1, hc) * LOG2E,
        a_dst21.reshape(1, hc) * LOG2E,
        a_src12 * LOG2E, a_dst12 * LOG2E, a_src22 * LOG2E, a_dst22 * LOG2E,
        b11.reshape(1, hc), b21.reshape(1, hc),
        b12.reshape(1, out_ch), b22.reshape(1, out_ch),
        combine_weight[0].T, combine_weight2[0].T,
    ], axis=0)

    full = lambda shape: pl.BlockSpec(shape, lambda p, i: tuple(0 for _ in shape))
    bias_rows = pl.BlockSpec((BLK, n), lambda p, i: ((1 - p) * i, 0))
    bf16 = jnp.bfloat16

    return pl.pallas_call(
        lambda *refs: _mega_kernel(heads, mid, out_ch, *refs),
        grid=(2, n // BLK),
        in_specs=[
            full((n, in_ch)),
            full((in_ch, hc)),
            full((in_ch, hc)),
            full((hc, out_ch)),
            full((hc, out_ch)),
            bias_rows,
            bias_rows,
            full((16, hc)),
        ],
        out_specs=pl.BlockSpec((BLK, out_ch), lambda p, i: (i, 0)),
        out_shape=jax.ShapeDtypeStruct((n, out_ch), f32),
        scratch_shapes=[
            pltpu.VMEM((heads * BLK, n), bf16),
            pltpu.VMEM((n, n), bf16),
            pltpu.VMEM((n, n), bf16),
            pltpu.VMEM((n, 2 * hc), f32),
            pltpu.VMEM((n, 2 * heads), bf16),
            pltpu.VMEM((n, 2 * heads), bf16),
            pltpu.VMEM((2 * heads, n), bf16),
            pltpu.VMEM((2 * heads, n), bf16),
            pltpu.VMEM((n, hc + 1), bf16),
            pltpu.VMEM((n, hc + 1), bf16),
            pltpu.VMEM((n, 2), bf16),
            pltpu.VMEM((n, 2), bf16),
            pltpu.VMEM((2, n), bf16),
            pltpu.VMEM((2, n), bf16),
            pltpu.VMEM((n, out_ch + 1), bf16),
            pltpu.VMEM((n, out_ch + 1), bf16),
            pltpu.VMEM((n, 2 * heads), f32),
            pltpu.VMEM((n, 2), f32),
        ],
    )(node_feature, W11, W21, W12, W22,
      one_order_bias, two_order_bias, params)
